# baseline (speedup 1.0000x reference)
import functools

import jax
import jax.numpy as jnp
from jax import lax
from jax.experimental import pallas as pl
from jax.experimental.pallas import tpu as pltpu
from jax.experimental.pallas import tpu_sc as plsc

_N = 1024
_C = 81
_HH = 28 * 28
_NW = 32
_ROWS = _N // _NW
_CW = 128
_CHUNKS = _ROWS * _HH // _CW


def _sc_gather_body(pm, idx3, dfl, idx2, yp, pd, idx_v, out_v, idx2_v, pd_v,
                    sem, sem2):
    c = lax.axis_index("c")
    s = lax.axis_index("s")
    wid = s * 2 + c
    r0 = wid * _ROWS

    pltpu.sync_copy(idx2.at[wid], idx2_v)
    pltpu.async_copy(dfl.at[idx2_v], pd_v, sem2)

    pltpu.sync_copy(idx3.at[wid], idx_v)

    def fire(j, carry):
        pltpu.async_copy(pm.at[idx_v.at[j]], out_v.at[j], sem)
        return carry

    lax.fori_loop(0, _CHUNKS, fire, 0)

    pltpu.make_async_copy(pm.at[pl.ds(0, _CW)], pd_v, sem2).wait()
    pltpu.sync_copy(pd_v, pd.at[wid])

    def drain(j, carry):
        pltpu.make_async_copy(pm.at[pl.ds(0, _CW)], out_v.at[j], sem).wait()
        return carry

    lax.fori_loop(0, _CHUNKS, drain, 0)
    pltpu.sync_copy(out_v, yp.at[wid])


@jax.jit
def _sc_gather(pm_flat, idx3, dfl, idx2):
    mesh = plsc.VectorSubcoreMesh(core_axis_name="c", subcore_axis_name="s")
    return pl.kernel(
        _sc_gather_body,
        out_type=(
            jax.ShapeDtypeStruct((_NW, _CHUNKS, _CW), jnp.float32),
            jax.ShapeDtypeStruct((_NW, _CW), jnp.float32),
        ),
        mesh=mesh,
        scratch_types=(
            pltpu.VMEM((_CHUNKS, _CW), jnp.int32),
            pltpu.VMEM((_CHUNKS, _CW), jnp.float32),
            pltpu.VMEM((_CW,), jnp.int32),
            pltpu.VMEM((_CW,), jnp.float32),
            pltpu.SemaphoreType.DMA,
            pltpu.SemaphoreType.DMA,
        ),
        compiler_params=pltpu.CompilerParams(use_tc_tiling_on_sc=False),
    )(pm_flat, idx3, dfl, idx2)


_BLK = 128


def _tc_loss_body(lg, cl, td, pd, tm, yp, out, acc):
    i = pl.program_id(0)

    @pl.when(i == 0)
    def _init():
        for k in range(5):
            acc[k] = 0.0

    cls = cl[...]
    keep = (cls != -1).astype(jnp.float32)
    pos = (cls > 0).astype(jnp.float32)
    safe = jnp.maximum(cls, 0)

    logits = lg[...]
    m = jnp.max(logits, axis=1, keepdims=True)
    lse = m + jnp.log(jnp.sum(jnp.exp(logits - m), axis=1, keepdims=True))
    iota = lax.broadcasted_iota(jnp.int32, logits.shape, 1)
    picked = jnp.sum(jnp.where(iota == safe, logits, 0.0), axis=1,
                     keepdims=True)
    ce = lse - picked
    acc[0] += jnp.sum(keep * ce)
    acc[1] += jnp.sum(keep)

    diff = jnp.abs(td[...] - pd[...])
    lt = (diff < 1.0).astype(jnp.float32)
    sl1 = lt * 0.5 * diff * diff + (1.0 - lt) * (diff - 0.5)
    acc[2] += jnp.sum(pos * sl1)
    acc[3] += jnp.sum(pos)

    ypv = yp[...]
    sce = (jnp.maximum(ypv, 0.0) - ypv * tm[...]
           + jnp.log1p(jnp.exp(-jnp.abs(ypv))))
    acc[4] += jnp.sum(pos * sce)

    @pl.when(i == pl.num_programs(0) - 1)
    def _finalize():
        total = (acc[0] / acc[1]
                 + acc[2] / (acc[3] * 4.0)
                 + acc[4] / (acc[3] * float(_HH)))
        out[...] = jnp.full((1, 1), total, dtype=jnp.float32)


def _tc_losses(logits, cls2, td, pd, tm, yp, interpret=False):
    grid = (_N // _BLK,)
    return pl.pallas_call(
        _tc_loss_body,
        grid=grid,
        in_specs=[
            pl.BlockSpec((_BLK, _C), lambda i: (i, 0)),
            pl.BlockSpec((_BLK, 1), lambda i: (i, 0)),
            pl.BlockSpec((_BLK, 4), lambda i: (i, 0)),
            pl.BlockSpec((_BLK, 4), lambda i: (i, 0)),
            pl.BlockSpec((_BLK, _HH), lambda i: (i, 0)),
            pl.BlockSpec((_BLK, _HH), lambda i: (i, 0)),
        ],
        out_specs=pl.BlockSpec((1, 1), lambda i: (0, 0)),
        out_shape=jax.ShapeDtypeStruct((1, 1), jnp.float32),
        scratch_shapes=[pltpu.SMEM((8,), jnp.float32)],
        interpret=interpret,
    )(logits, cls2, td, pd, tm, yp)


def kernel(target_deltas, mrcnn_deltas, mrcnn_class_logits, target_masks,
           pred_masks, target_class_ids):
    cls = target_class_ids.astype(jnp.int32)
    safe = jnp.maximum(cls, 0)

    rows = jnp.arange(_N, dtype=jnp.int32)
    base = rows * (_HH * _C) + safe
    ramp = jnp.arange(_HH, dtype=jnp.int32) * _C
    idx3 = (base[:, None] + ramp[None, :]).reshape(_NW, _CHUNKS, _CW)
    idx2 = (((rows * _C + safe) * 4)[:, None]
            + jnp.arange(4, dtype=jnp.int32)[None, :]).reshape(_NW, _CW)

    yp3, pd2 = _sc_gather(
        pred_masks.reshape(-1),
        idx3,
        mrcnn_deltas.reshape(-1),
        idx2,
    )
    yp = yp3.reshape(_N, _HH)
    pd = pd2.reshape(_N, 4)
    tm = target_masks.reshape(_N, _HH)

    out = _tc_losses(mrcnn_class_logits, cls[:, None], target_deltas, pd,
                     tm, yp)
    return out[0, 0]
